# baseline (speedup 1.0000x reference)
import math

import jax
import jax.numpy as jnp
from jax import lax
from jax.experimental import pallas as pl
from jax.experimental.pallas import tpu as pltpu
from jax.experimental.pallas import tpu_sc as plsc

_N = 10000
_DEG = 32
_D = 128
_DELTA = 3.4965

_NC = 2
_NS = 16
_NW = _NC * _NS

_NB = 8
_FG = _D // 16

_N_TC = 7600
_N_SC = _N - _N_TC

_C1 = math.log(_DEG + 1) / _DELTA
_C2 = _DELTA / math.log(_DEG + 1)


def _w_eff(w_ref):
    w = w_ref[...]
    return (
        w[0 : 4 * _D, :]
        + _C1 * w[4 * _D : 8 * _D, :]
        + _C2 * w[8 * _D : 12 * _D, :]
    )


_BN = 760
_NSTEPS = _N_TC // _BN


def _tc_fused_body(x_hbm, w_ref, b_ref, o_ref, vbuf, sems):
    i = pl.program_id(0)

    def copies(step, slot):
        return [
            pltpu.make_async_copy(
                x_hbm.at[pl.ds(step * _BN, _BN), d],
                vbuf.at[slot, d],
                sems.at[slot],
            )
            for d in range(_DEG)
        ]

    @pl.when(i == 0)
    def _():
        for c in copies(0, 0):
            c.start()

    @pl.when(i + 1 < _NSTEPS)
    def _():
        for c in copies(i + 1, (i + 1) % 2):
            c.start()

    def compute(slot):
        v0 = vbuf[slot, 0]
        s = v0
        sq = v0 * v0
        mx = v0
        mn = v0
        for d in range(1, _DEG):
            v = vbuf[slot, d]
            s = s + v
            sq = sq + v * v
            mx = jnp.maximum(mx, v)
            mn = jnp.minimum(mn, v)
        mean = s * (1.0 / _DEG)
        var = sq * (1.0 / _DEG) - mean * mean
        std = jnp.sqrt(jnp.maximum(var, 0.0))
        we = _w_eff(w_ref)
        acc = jnp.dot(mean, we[0 * _D : 1 * _D, :])
        acc += jnp.dot(mx, we[1 * _D : 2 * _D, :])
        acc += jnp.dot(mn, we[2 * _D : 3 * _D, :])
        acc += jnp.dot(std, we[3 * _D : 4 * _D, :])
        o_ref[...] = acc + b_ref[...]

    @pl.when(i % 2 == 0)
    def _():
        for c in copies(i, 0):
            c.wait()
        compute(0)

    @pl.when(i % 2 == 1)
    def _():
        for c in copies(i, 1):
            c.wait()
        compute(1)


def _tc_fused(x, W, b2):
    return pl.pallas_call(
        _tc_fused_body,
        grid=(_NSTEPS,),
        in_specs=[
            pl.BlockSpec(memory_space=pl.ANY),
            pl.BlockSpec((12 * _D, _D), lambda i: (0, 0)),
            pl.BlockSpec((1, _D), lambda i: (0, 0)),
        ],
        out_specs=pl.BlockSpec((_BN, _D), lambda i: (i, 0)),
        out_shape=jax.ShapeDtypeStruct((_N, _D), jnp.float32),
        scratch_shapes=[
            pltpu.VMEM((2, _DEG, _BN, _D), jnp.float32),
            pltpu.SemaphoreType.DMA((2,)),
        ],
    )(x, W, b2)


_SC_CHUNK0 = _N_TC // _NB
_SC_NCHUNKS = _N_SC // _NB
_SC_T = -(-_SC_NCHUNKS // _NW)


def _sc_body(x_hbm, s_hbm, buf0, buf1, outb0, outb1,
             sem_i0, sem_i1, sem_o0, sem_o1):
    w = lax.axis_index("s") * _NC + lax.axis_index("c")
    c0 = w * _SC_T
    nch = jnp.minimum(_SC_T, jnp.maximum(0, _SC_NCHUNKS - c0))

    def in_copy(t, buf, sem):
        c = _SC_CHUNK0 + c0 + t
        return pltpu.make_async_copy(x_hbm.at[pl.ds(c * _NB, _NB)], buf, sem)

    def out_copy(t, outb, sem):
        c = c0 + t
        return pltpu.make_async_copy(outb, s_hbm.at[pl.ds(c * _NB, _NB)], sem)

    @pl.when(nch > 0)
    def _():
        in_copy(0, buf0, sem_i0).start()

    @pl.when(nch > 1)
    def _():
        in_copy(1, buf1, sem_i1).start()

    def compute(buf, outb):
        def node_body(n, carry):
            s = [buf[n, 0, pl.ds(16 * f, 16)] for f in range(_FG)]
            mx = list(s)
            mn = list(s)
            sq = [v * v for v in s]
            for d in range(1, _DEG):
                for f in range(_FG):
                    v = buf[n, d, pl.ds(16 * f, 16)]
                    s[f] = s[f] + v
                    sq[f] = sq[f] + v * v
                    mx[f] = jnp.maximum(mx[f], v)
                    mn[f] = jnp.minimum(mn[f], v)
            for f in range(_FG):
                outb[n, pl.ds(16 * f, 16)] = s[f]
                outb[n, pl.ds(_D + 16 * f, 16)] = mx[f]
                outb[n, pl.ds(2 * _D + 16 * f, 16)] = mn[f]
                outb[n, pl.ds(3 * _D + 16 * f, 16)] = sq[f]
            return carry

        lax.fori_loop(0, _NB, node_body, 0)

    def process(t, buf, outb, sem_i, sem_o):
        in_copy(t, buf, sem_i).wait()

        @pl.when(t >= 2)
        def _():
            out_copy(t - 2, outb, sem_o).wait()

        compute(buf, outb)
        out_copy(t, outb, sem_o).start()

        @pl.when(t + 2 < nch)
        def _():
            in_copy(t + 2, buf, sem_i).start()

    def loop_body(t, carry):
        @pl.when(t % 2 == 0)
        def _():
            process(t, buf0, outb0, sem_i0, sem_o0)

        @pl.when(t % 2 == 1)
        def _():
            process(t, buf1, outb1, sem_i1, sem_o1)

        return carry

    lax.fori_loop(0, nch, loop_body, 0)

    last = nch - 1

    @pl.when(jnp.logical_and(nch >= 1, last % 2 == 0))
    def _():
        out_copy(last, outb0, sem_o0).wait()

    @pl.when(jnp.logical_and(nch >= 1, last % 2 == 1))
    def _():
        out_copy(last, outb1, sem_o1).wait()

    @pl.when(jnp.logical_and(nch >= 2, last % 2 == 0))
    def _():
        out_copy(last - 1, outb1, sem_o1).wait()

    @pl.when(jnp.logical_and(nch >= 2, last % 2 == 1))
    def _():
        out_copy(last - 1, outb0, sem_o0).wait()


def _sc_aggregate(x):
    mesh = plsc.VectorSubcoreMesh(
        core_axis_name="c", subcore_axis_name="s",
        num_cores=_NC, num_subcores=_NS,
    )
    fn = pl.kernel(
        _sc_body,
        out_type=jax.ShapeDtypeStruct((_N_SC, 4 * _D), jnp.float32),
        mesh=mesh,
        scratch_types=[
            pltpu.VMEM((_NB, _DEG, _D), jnp.float32),
            pltpu.VMEM((_NB, _DEG, _D), jnp.float32),
            pltpu.VMEM((_NB, 4 * _D), jnp.float32),
            pltpu.VMEM((_NB, 4 * _D), jnp.float32),
            pltpu.SemaphoreType.DMA,
            pltpu.SemaphoreType.DMA,
            pltpu.SemaphoreType.DMA,
            pltpu.SemaphoreType.DMA,
        ],
    )
    return fn(x)


_BN2 = 400


def _tc_finish_body(s_ref, w_ref, b_ref, _o_alias_ref, o_ref):
    sb = s_ref[...]
    ssum = sb[:, 0 * _D : 1 * _D]
    mx = sb[:, 1 * _D : 2 * _D]
    mn = sb[:, 2 * _D : 3 * _D]
    ssq = sb[:, 3 * _D : 4 * _D]
    mean = ssum * (1.0 / _DEG)
    var = ssq * (1.0 / _DEG) - mean * mean
    std = jnp.sqrt(jnp.maximum(var, 0.0))
    we = _w_eff(w_ref)
    acc = jnp.dot(mean, we[0 * _D : 1 * _D, :])
    acc += jnp.dot(mx, we[1 * _D : 2 * _D, :])
    acc += jnp.dot(mn, we[2 * _D : 3 * _D, :])
    acc += jnp.dot(std, we[3 * _D : 4 * _D, :])
    o_ref[...] = acc + b_ref[...]


def _tc_finish(S, W, b2, out_partial):
    off = _N_TC // _BN2
    return pl.pallas_call(
        _tc_finish_body,
        grid=(_N_SC // _BN2,),
        in_specs=[
            pl.BlockSpec((_BN2, 4 * _D), lambda i: (i, 0)),
            pl.BlockSpec((12 * _D, _D), lambda i: (0, 0)),
            pl.BlockSpec((1, _D), lambda i: (0, 0)),
            pl.BlockSpec(memory_space=pl.ANY),
        ],
        out_specs=pl.BlockSpec((_BN2, _D), lambda i, _o=off: (i + _o, 0)),
        out_shape=jax.ShapeDtypeStruct((_N, _D), jnp.float32),
        input_output_aliases={3: 0},
    )(S, W, b2, out_partial)


def kernel(x, W, b):
    b2 = b.reshape(1, _D)
    S = _sc_aggregate(x)
    out_a = _tc_fused(x, W, b2)
    return _tc_finish(S, W, b2, out_a)


# speedup vs baseline: 1.4229x; 1.4229x over previous
import math

import jax
import jax.numpy as jnp
from jax.experimental import pallas as pl
from jax.experimental.pallas import tpu as pltpu

_N = 10000
_DEG = 32
_D = 128
_DELTA = 3.4965
_BN = 1000
_NSTEPS = _N // _BN

_C1 = math.log(_DEG + 1) / _DELTA
_C2 = _DELTA / math.log(_DEG + 1)


def _pna_kernel(x_hbm, w_ref, b_ref, o_ref, vbuf, sems):
    i = pl.program_id(0)

    def copies(step, slot):
        return [
            pltpu.make_async_copy(
                x_hbm.at[pl.ds(step * _BN, _BN), d],
                vbuf.at[slot, d],
                sems.at[slot],
            )
            for d in range(_DEG)
        ]

    @pl.when(i == 0)
    def _():
        for c in copies(0, 0):
            c.start()

    @pl.when(i + 1 < _NSTEPS)
    def _():
        for c in copies(i + 1, (i + 1) % 2):
            c.start()

    def compute(slot):
        v0 = vbuf[slot, 0]
        s = v0
        sq = v0 * v0
        mx = v0
        mn = v0
        for d in range(1, _DEG):
            v = vbuf[slot, d]
            s = s + v
            sq = sq + v * v
            mx = jnp.maximum(mx, v)
            mn = jnp.minimum(mn, v)

        mean = s * (1.0 / _DEG)
        var = sq * (1.0 / _DEG) - mean * mean
        std = jnp.sqrt(jnp.maximum(var, 0.0))

        w = w_ref[...]
        we = (
            w[0 : 4 * _D, :]
            + _C1 * w[4 * _D : 8 * _D, :]
            + _C2 * w[8 * _D : 12 * _D, :]
        )
        acc = jnp.dot(mean, we[0 * _D : 1 * _D, :])
        acc += jnp.dot(mx, we[1 * _D : 2 * _D, :])
        acc += jnp.dot(mn, we[2 * _D : 3 * _D, :])
        acc += jnp.dot(std, we[3 * _D : 4 * _D, :])
        o_ref[...] = acc + b_ref[...]

    @pl.when(i % 2 == 0)
    def _():
        for c in copies(i, 0):
            c.wait()
        compute(0)

    @pl.when(i % 2 == 1)
    def _():
        for c in copies(i, 1):
            c.wait()
        compute(1)


def kernel(x, W, b):
    b2 = b.reshape(1, _D)
    return pl.pallas_call(
        _pna_kernel,
        grid=(_NSTEPS,),
        in_specs=[
            pl.BlockSpec(memory_space=pl.ANY),
            pl.BlockSpec((12 * _D, _D), lambda i: (0, 0)),
            pl.BlockSpec((1, _D), lambda i: (0, 0)),
        ],
        out_specs=pl.BlockSpec((_BN, _D), lambda i: (i, 0)),
        out_shape=jax.ShapeDtypeStruct((_N, _D), jnp.float32),
        scratch_shapes=[
            pltpu.VMEM((2, _DEG, _BN, _D), jnp.float32),
            pltpu.SemaphoreType.DMA((2,)),
        ],
    )(x, W, b2)
